# Initial kernel scaffold; baseline (speedup 1.0000x reference)
import functools

import jax
import jax.numpy as jnp
from jax import lax
from jax.experimental import pallas as pl
from jax.experimental.pallas import tpu as pltpu
from jax.experimental.pallas import tpu_sc as plsc

N = 4096
H = 64
K = 32
WIN = 6
WW = 2 * WIN + 1
NCAND = WW * WW
NCPAD = 176
PAD = WIN * H + WIN
NKNOT = 32


def _select_body(cxp_ref, cyp_ref, knots_ref, smx_ref, smy_ref,
                 phiw_ref, phii_ref, phij_ref, j_ref, w_ref):
    f32 = jnp.float32
    cx = cxp_ref[:, PAD:PAD + N]
    cy = cyp_ref[:, PAD:PAD + N]
    p = lax.broadcasted_iota(jnp.int32, (1, N), 1)
    r = p // H
    c = p % H

    rows = []
    inf_row = jnp.full((1, N), jnp.inf, dtype=f32)
    for o in range(NCAND):
        dr = o // WW - WIN
        dc = o % WW - WIN
        s = dr * H + dc
        cxs = cxp_ref[:, PAD + s:PAD + s + N]
        cys = cyp_ref[:, PAD + s:PAD + s + N]
        dx = cx - cxs
        dy = cy - cys
        dist = jnp.sqrt(dx * dx + dy * dy)
        u = r + dr
        v = c + dc
        ok = (u >= 0) & (u < H) & (v >= 0) & (v < H)
        rows.append(jnp.where(ok, dist, jnp.inf))
    for _ in range(NCPAD - NCAND):
        rows.append(inf_row)
    D = jnp.concatenate(rows, axis=0)
    ORD = lax.broadcasted_iota(jnp.int32, (NCPAD, N), 0)

    js = []
    for _ in range(K):
        m = jnp.min(D, axis=0, keepdims=True)
        wo = jnp.min(jnp.where(D == m, ORD, NCPAD), axis=0, keepdims=True)
        dr = wo // WW - WIN
        dc = wo % WW - WIN
        js.append(p + dr * H + dc)
        D = jnp.where(ORD == wo, jnp.inf, D)
    J = jnp.concatenate(js, axis=0)

    s390 = J - p + PAD
    DR = s390 // H - WIN
    DC = s390 % H - WIN
    inv = f32(1.0 / (H - 1))
    relx = -DR.astype(f32) * inv
    rely = -DC.astype(f32) * inv

    psix = jnp.zeros((K, N), dtype=f32)
    psiy = jnp.zeros((K, N), dtype=f32)
    for t in range(NKNOT):
        kt = knots_ref[0, t]
        psix = psix + jnp.maximum(0.0, 1.0 - jnp.abs(relx - kt)) * smx_ref[0, t]
        psiy = psiy + jnp.maximum(0.0, 1.0 - jnp.abs(rely - kt)) * smy_ref[0, t]
    psi = psix * psiy

    pw = phiw_ref[0:1, :]
    m00 = jnp.sum(pw * phii_ref[0:1, :] * phij_ref[0:1, :])
    m01 = jnp.sum(pw * phii_ref[0:1, :] * phij_ref[1:2, :])
    m10 = jnp.sum(pw * phii_ref[1:2, :] * phij_ref[0:1, :])
    m11 = jnp.sum(pw * phii_ref[1:2, :] * phij_ref[1:2, :])
    cjx = cx + DR.astype(f32) * inv
    cjy = cy + DC.astype(f32) * inv
    q = cx * (m00 * cjx + m01 * cjy) + cy * (m10 * cjx + m11 * cjy)

    nedge = f32(1.0 / (K * N))
    mp = jnp.sum(jnp.abs(psi)) * nedge + 1e-6
    mq = jnp.sum(jnp.abs(q)) * nedge + 1e-6

    j_ref[...] = J
    w_ref[...] = (psi / mp) * (q / mq)


def _select_weights(cxp, cyp, knots, smx, smy, phiw, phii2, phij2):
    return pl.pallas_call(
        _select_body,
        out_shape=(
            jax.ShapeDtypeStruct((K, N), jnp.int32),
            jax.ShapeDtypeStruct((K, N), jnp.float32),
        ),
    )(cxp, cyp, knots, smx, smy, phiw, phii2, phij2)


def _nbr_body(xf_hbm, jg_hbm, wg_hbm, out_hbm, idx_v, w_v, rows_v, acc_v, sem,
              *, nper, chunk):
    wid = lax.axis_index("c") * 16 + lax.axis_index("s")
    base = wid * nper
    ek = chunk * K

    def do_chunk(ci, _):
        row0 = base + ci * chunk
        e0 = row0 * K
        pltpu.sync_copy(jg_hbm.at[pl.ds(e0, ek)], idx_v)
        pltpu.sync_copy(wg_hbm.at[pl.ds(e0, ek)], w_v)
        pltpu.async_copy(xf_hbm.at[idx_v], rows_v, sem).wait()
        for g in range(chunk):
            def kbody(kk, acc):
                e = g * K + kk
                s = w_v[e]
                return tuple(acc[v] + s * rows_v[e, pl.ds(v * 16, 16)]
                             for v in range(8))
            acc = lax.fori_loop(
                0, K, kbody,
                tuple(jnp.zeros((16,), jnp.float32) for _ in range(8)))
            for v in range(8):
                acc_v[g, pl.ds(v * 16, 16)] = acc[v]
        pltpu.sync_copy(acc_v, out_hbm.at[pl.ds(row0, chunk)])
        return 0

    lax.fori_loop(0, nper // chunk, do_chunk, 0)


def _nbr_sum(xf, jg, wg):
    rows, cdim = xf.shape
    nw = 32
    nper = rows // nw
    chunk = 8
    mesh = plsc.VectorSubcoreMesh(core_axis_name="c", subcore_axis_name="s")
    fn = functools.partial(
        pl.kernel,
        mesh=mesh,
        out_type=jax.ShapeDtypeStruct((rows, cdim), jnp.float32),
        scratch_types=[
            pltpu.VMEM((chunk * K,), jnp.int32),
            pltpu.VMEM((chunk * K,), jnp.float32),
            pltpu.VMEM((chunk * K, cdim), jnp.float32),
            pltpu.VMEM((chunk, cdim), jnp.float32),
            pltpu.SemaphoreType.DMA,
        ],
    )(functools.partial(_nbr_body, nper=nper, chunk=chunk))
    return fn(xf, jg, wg)


def _mlp_body(x_ref, nbr_ref, w1_ref, b1_ref, w2_ref, b2_ref, out_ref):
    f32 = jnp.float32
    h = lax.dot_general(x_ref[...], w1_ref[...],
                        (((1,), (1,)), ((), ())),
                        preferred_element_type=f32)
    h = jnp.maximum(h + b1_ref[...], 0.0)
    y = lax.dot_general(h, w2_ref[...],
                        (((1,), (1,)), ((), ())),
                        preferred_element_type=f32)
    out_ref[...] = y + b2_ref[...] + nbr_ref[...] * (1.0 / K)


def _mlp_add(xf, nbr, w1, b1, w2, b2):
    rows, cdim = xf.shape
    blk = 1024
    grid = rows // blk
    return pl.pallas_call(
        _mlp_body,
        grid=(grid,),
        in_specs=[
            pl.BlockSpec((blk, cdim), lambda i: (i, 0)),
            pl.BlockSpec((blk, cdim), lambda i: (i, 0)),
            pl.BlockSpec(w1.shape, lambda i: (0, 0)),
            pl.BlockSpec(b1.shape, lambda i: (0, 0)),
            pl.BlockSpec(w2.shape, lambda i: (0, 0)),
            pl.BlockSpec(b2.shape, lambda i: (0, 0)),
        ],
        out_specs=pl.BlockSpec((blk, cdim), lambda i: (i, 0)),
        out_shape=jax.ShapeDtypeStruct((rows, cdim), jnp.float32),
    )(xf, nbr, w1, b1, w2, b2)


def kernel(x, W1_w, W1_b, W2_w, W2_b, phi_w, phi_i, phi_j,
           h1_w, h1_b, h2_w, h2_b, S_m_x, S_m_y):
    B, n, cdim = x.shape
    lin = jnp.linspace(0.0, 1.0, H)
    cx = jnp.repeat(lin, H).reshape(1, N)
    cy = jnp.tile(lin, H).reshape(1, N)
    zpad = jnp.zeros((1, PAD), dtype=jnp.float32)
    cxp = jnp.concatenate([zpad, cx, zpad], axis=1)
    cyp = jnp.concatenate([zpad, cy, zpad], axis=1)
    knots = jnp.linspace(0.0, 1.0, NKNOT).reshape(1, NKNOT)

    J, W = _select_weights(
        cxp, cyp, knots,
        S_m_x.reshape(1, NKNOT), S_m_y.reshape(1, NKNOT),
        phi_w.reshape(1, cdim), phi_i.T, phi_j.T)

    jf = J.T.reshape(-1)
    wf = W.T.reshape(-1)
    jg = jnp.concatenate([jf, jf + N])
    wg = jnp.concatenate([wf, wf])

    xf = x.reshape(B * n, cdim)
    nbr = _nbr_sum(xf, jg, wg)
    out = _mlp_add(xf, nbr, W1_w, W1_b.reshape(1, 2 * cdim),
                   W2_w, W2_b.reshape(1, cdim))
    return out.reshape(B, n, cdim)


# speedup vs baseline: 80.8893x; 80.8893x over previous
import functools

import jax
import jax.numpy as jnp
from jax import lax
from jax.experimental import pallas as pl
from jax.experimental.pallas import tpu as pltpu
from jax.experimental.pallas import tpu_sc as plsc

N = 4096
H = 64
K = 32
WIN = 6
WW = 2 * WIN + 1
NCAND = WW * WW
NCPAD = 176
PAD = WIN * H + WIN
NKNOT = 32


def _select_body(cxp_ref, cyp_ref, knots_ref, smx_ref, smy_ref,
                 phiw_ref, phii_ref, phij_ref, j_ref, w_ref):
    f32 = jnp.float32
    cx = cxp_ref[:, PAD:PAD + N]
    cy = cyp_ref[:, PAD:PAD + N]
    p = lax.broadcasted_iota(jnp.int32, (1, N), 1)
    r = p // H
    c = p % H

    rows = []
    inf_row = jnp.full((1, N), jnp.inf, dtype=f32)
    for o in range(NCAND):
        dr = o // WW - WIN
        dc = o % WW - WIN
        s = dr * H + dc
        cxs = cxp_ref[:, PAD + s:PAD + s + N]
        cys = cyp_ref[:, PAD + s:PAD + s + N]
        dx = cx - cxs
        dy = cy - cys
        dist = jnp.sqrt(dx * dx + dy * dy)
        u = r + dr
        v = c + dc
        ok = (u >= 0) & (u < H) & (v >= 0) & (v < H)
        rows.append(jnp.where(ok, dist, jnp.inf))
    for _ in range(NCPAD - NCAND):
        rows.append(inf_row)
    D = jnp.concatenate(rows, axis=0)
    ORD = lax.broadcasted_iota(jnp.int32, (NCPAD, N), 0)

    js = []
    for _ in range(K):
        m = jnp.min(D, axis=0, keepdims=True)
        wo = jnp.min(jnp.where(D == m, ORD, NCPAD), axis=0, keepdims=True)
        dr = wo // WW - WIN
        dc = wo % WW - WIN
        js.append(p + dr * H + dc)
        D = jnp.where(ORD == wo, jnp.inf, D)
    J = jnp.concatenate(js, axis=0)

    s390 = J - p + PAD
    DR = s390 // H - WIN
    DC = s390 % H - WIN
    inv = f32(1.0 / (H - 1))
    relx = -DR.astype(f32) * inv
    rely = -DC.astype(f32) * inv

    psix = jnp.zeros((K, N), dtype=f32)
    psiy = jnp.zeros((K, N), dtype=f32)
    for t in range(NKNOT):
        kt = knots_ref[0, t]
        psix = psix + jnp.maximum(0.0, 1.0 - jnp.abs(relx - kt)) * smx_ref[0, t]
        psiy = psiy + jnp.maximum(0.0, 1.0 - jnp.abs(rely - kt)) * smy_ref[0, t]
    psi = psix * psiy

    pw = phiw_ref[0:1, :]
    m00 = jnp.sum(pw * phii_ref[0:1, :] * phij_ref[0:1, :])
    m01 = jnp.sum(pw * phii_ref[0:1, :] * phij_ref[1:2, :])
    m10 = jnp.sum(pw * phii_ref[1:2, :] * phij_ref[0:1, :])
    m11 = jnp.sum(pw * phii_ref[1:2, :] * phij_ref[1:2, :])
    cjx = cx + DR.astype(f32) * inv
    cjy = cy + DC.astype(f32) * inv
    q = cx * (m00 * cjx + m01 * cjy) + cy * (m10 * cjx + m11 * cjy)

    nedge = f32(1.0 / (K * N))
    mp = jnp.sum(jnp.abs(psi)) * nedge + 1e-6
    mq = jnp.sum(jnp.abs(q)) * nedge + 1e-6

    j_ref[...] = J
    w_ref[...] = (psi / mp) * (q / mq)


def _select_weights(cxp, cyp, knots, smx, smy, phiw, phii2, phij2):
    return pl.pallas_call(
        _select_body,
        out_shape=(
            jax.ShapeDtypeStruct((K, N), jnp.int32),
            jax.ShapeDtypeStruct((K, N), jnp.float32),
        ),
    )(cxp, cyp, knots, smx, smy, phiw, phii2, phij2)


def _nbr_body(xf_hbm, jg_hbm, wg_hbm, out_hbm, idx_v, w_v, rows_v, acc_v, sem,
              *, nper, chunk):
    wid = lax.axis_index("c") * 16 + lax.axis_index("s")
    base = wid * nper
    ek = chunk * K

    def do_chunk(ci, _):
        row0 = base + ci * chunk
        e0 = row0 * K
        pltpu.sync_copy(jg_hbm.at[pl.ds(e0, ek)], idx_v)
        pltpu.sync_copy(wg_hbm.at[pl.ds(e0, ek)], w_v.at[pl.ds(0, ek)])
        pltpu.async_copy(xf_hbm.at[idx_v], rows_v, sem).wait()
        for g in range(chunk):
            def kbody(kk, acc):
                e = g * K + kk
                s = w_v[pl.ds(e, 16)][0]
                return tuple(acc[v] + s * rows_v[e, pl.ds(v * 16, 16)]
                             for v in range(8))
            acc = lax.fori_loop(
                0, K, kbody,
                tuple(jnp.zeros((16,), jnp.float32) for _ in range(8)))
            for v in range(8):
                acc_v[g, pl.ds(v * 16, 16)] = acc[v]
        pltpu.sync_copy(acc_v, out_hbm.at[pl.ds(row0, chunk)])
        return 0

    lax.fori_loop(0, nper // chunk, do_chunk, 0)


def _nbr_sum(xf, jg, wg):
    rows, cdim = xf.shape
    nw = 32
    nper = rows // nw
    chunk = 8
    mesh = plsc.VectorSubcoreMesh(core_axis_name="c", subcore_axis_name="s")
    fn = functools.partial(
        pl.kernel,
        mesh=mesh,
        out_type=jax.ShapeDtypeStruct((rows, cdim), jnp.float32),
        scratch_types=[
            pltpu.VMEM((chunk * K,), jnp.int32),
            pltpu.VMEM((chunk * K + 16,), jnp.float32),
            pltpu.VMEM((chunk * K, cdim), jnp.float32),
            pltpu.VMEM((chunk, cdim), jnp.float32),
            pltpu.SemaphoreType.DMA,
        ],
    )(functools.partial(_nbr_body, nper=nper, chunk=chunk))
    return fn(xf, jg, wg)


def _mlp_body(x_ref, nbr_ref, w1_ref, b1_ref, w2_ref, b2_ref, out_ref):
    f32 = jnp.float32
    h = lax.dot_general(x_ref[...], w1_ref[...],
                        (((1,), (1,)), ((), ())),
                        preferred_element_type=f32)
    h = jnp.maximum(h + b1_ref[...], 0.0)
    y = lax.dot_general(h, w2_ref[...],
                        (((1,), (1,)), ((), ())),
                        preferred_element_type=f32)
    out_ref[...] = y + b2_ref[...] + nbr_ref[...] * (1.0 / K)


def _mlp_add(xf, nbr, w1, b1, w2, b2):
    rows, cdim = xf.shape
    blk = 1024
    grid = rows // blk
    return pl.pallas_call(
        _mlp_body,
        grid=(grid,),
        in_specs=[
            pl.BlockSpec((blk, cdim), lambda i: (i, 0)),
            pl.BlockSpec((blk, cdim), lambda i: (i, 0)),
            pl.BlockSpec(w1.shape, lambda i: (0, 0)),
            pl.BlockSpec(b1.shape, lambda i: (0, 0)),
            pl.BlockSpec(w2.shape, lambda i: (0, 0)),
            pl.BlockSpec(b2.shape, lambda i: (0, 0)),
        ],
        out_specs=pl.BlockSpec((blk, cdim), lambda i: (i, 0)),
        out_shape=jax.ShapeDtypeStruct((rows, cdim), jnp.float32),
    )(xf, nbr, w1, b1, w2, b2)


def kernel(x, W1_w, W1_b, W2_w, W2_b, phi_w, phi_i, phi_j,
           h1_w, h1_b, h2_w, h2_b, S_m_x, S_m_y):
    B, n, cdim = x.shape
    lin = jnp.linspace(0.0, 1.0, H)
    cx = jnp.repeat(lin, H).reshape(1, N)
    cy = jnp.tile(lin, H).reshape(1, N)
    zpad = jnp.zeros((1, PAD), dtype=jnp.float32)
    cxp = jnp.concatenate([zpad, cx, zpad], axis=1)
    cyp = jnp.concatenate([zpad, cy, zpad], axis=1)
    knots = jnp.linspace(0.0, 1.0, NKNOT).reshape(1, NKNOT)

    J, W = _select_weights(
        cxp, cyp, knots,
        S_m_x.reshape(1, NKNOT), S_m_y.reshape(1, NKNOT),
        phi_w.reshape(1, cdim), phi_i.T, phi_j.T)

    jf = J.T.reshape(-1)
    wf = W.T.reshape(-1)
    jg = jnp.concatenate([jf, jf + N])
    wg = jnp.concatenate([wf, wf])

    xf = x.reshape(B * n, cdim)
    nbr = _nbr_sum(xf, jg, wg)
    out = _mlp_add(xf, nbr, W1_w, W1_b.reshape(1, 2 * cdim),
                   W2_w, W2_b.reshape(1, cdim))
    return out.reshape(B, n, cdim)


# speedup vs baseline: 126.4458x; 1.5632x over previous
import functools

import jax
import jax.numpy as jnp
from jax import lax
from jax.experimental import pallas as pl
from jax.experimental.pallas import tpu as pltpu
from jax.experimental.pallas import tpu_sc as plsc

N = 4096
H = 64
K = 32
WIN = 5
WW = 2 * WIN + 1
NCAND = WW * WW
NCPAD = 128
PAD = WIN * H + WIN
NKNOT = 32
HALO = 328
SLAB = 256 + 2 * HALO


def _select_body(cxp_ref, cyp_ref, knots_ref, smx_ref, smy_ref,
                 phiw_ref, phii_ref, phij_ref, j_ref, w_ref):
    f32 = jnp.float32
    cx = cxp_ref[:, PAD:PAD + N]
    cy = cyp_ref[:, PAD:PAD + N]
    p = lax.broadcasted_iota(jnp.int32, (1, N), 1)
    r = p // H
    c = p % H

    rows = []
    inf_row = jnp.full((1, N), jnp.inf, dtype=f32)
    for o in range(NCAND):
        dr = o // WW - WIN
        dc = o % WW - WIN
        s = dr * H + dc
        cxs = cxp_ref[:, PAD + s:PAD + s + N]
        cys = cyp_ref[:, PAD + s:PAD + s + N]
        dx = cx - cxs
        dy = cy - cys
        dist = jnp.sqrt(dx * dx + dy * dy)
        u = r + dr
        v = c + dc
        ok = (u >= 0) & (u < H) & (v >= 0) & (v < H)
        rows.append(jnp.where(ok, dist, jnp.inf))
    for _ in range(NCPAD - NCAND):
        rows.append(inf_row)
    D = jnp.concatenate(rows, axis=0)
    ORD = lax.broadcasted_iota(jnp.int32, (NCPAD, N), 0)

    js = []
    for _ in range(K):
        m = jnp.min(D, axis=0, keepdims=True)
        wo = jnp.min(jnp.where(D == m, ORD, NCPAD), axis=0, keepdims=True)
        dr = wo // WW - WIN
        dc = wo % WW - WIN
        js.append(p + dr * H + dc)
        D = jnp.where(ORD == wo, jnp.inf, D)
    J = jnp.concatenate(js, axis=0)

    s390 = J - p + PAD
    DR = s390 // H - WIN
    DC = s390 % H - WIN
    inv = f32(1.0 / (H - 1))
    relx = -DR.astype(f32) * inv
    rely = -DC.astype(f32) * inv

    psix = jnp.zeros((K, N), dtype=f32)
    psiy = jnp.zeros((K, N), dtype=f32)
    for t in range(NKNOT):
        kt = knots_ref[0, t]
        psix = psix + jnp.maximum(0.0, 1.0 - jnp.abs(relx - kt)) * smx_ref[0, t]
        psiy = psiy + jnp.maximum(0.0, 1.0 - jnp.abs(rely - kt)) * smy_ref[0, t]
    psi = psix * psiy

    pw = phiw_ref[0:1, :]
    m00 = jnp.sum(pw * phii_ref[0:1, :] * phij_ref[0:1, :])
    m01 = jnp.sum(pw * phii_ref[0:1, :] * phij_ref[1:2, :])
    m10 = jnp.sum(pw * phii_ref[1:2, :] * phij_ref[0:1, :])
    m11 = jnp.sum(pw * phii_ref[1:2, :] * phij_ref[1:2, :])
    cjx = cx + DR.astype(f32) * inv
    cjy = cy + DC.astype(f32) * inv
    q = cx * (m00 * cjx + m01 * cjy) + cy * (m10 * cjx + m11 * cjy)

    nedge = f32(1.0 / (K * N))
    mp = jnp.sum(jnp.abs(psi)) * nedge + 1e-6
    mq = jnp.sum(jnp.abs(q)) * nedge + 1e-6

    j_ref[...] = J
    w_ref[...] = (psi / mp) * (q / mq)


def _select_weights(cxp, cyp, knots, smx, smy, phiw, phii2, phij2):
    return pl.pallas_call(
        _select_body,
        out_shape=(
            jax.ShapeDtypeStruct((K, N), jnp.int32),
            jax.ShapeDtypeStruct((K, N), jnp.float32),
        ),
    )(cxp, cyp, knots, smx, smy, phiw, phii2, phij2)


def _nbr_body(xf_hbm, jg_hbm, wg_hbm, out_hbm, slab_v, idx_v, w_v, out_v,
              *, nper, chunk, nbatch):
    wid = lax.axis_index("c") * 16 + lax.axis_index("s")
    base = wid * nper
    p0 = lax.rem(base, nbatch)
    b0 = base - p0
    s0 = b0 + jnp.clip(p0 - HALO, 0, nbatch - SLAB)
    s0 = pl.multiple_of(s0, 8)
    pltpu.sync_copy(xf_hbm.at[pl.ds(s0, SLAB)], slab_v)
    ek = chunk * K

    def do_chunk(ci, _):
        row0 = base + ci * chunk
        e0 = row0 * K
        pltpu.sync_copy(jg_hbm.at[pl.ds(e0, ek)], idx_v)
        pltpu.sync_copy(wg_hbm.at[pl.ds(e0, ek)], w_v)

        def dest(g, _g):
            eg = g * K
            acc = [jnp.zeros((16,), jnp.float32) for _ in range(8)]
            for h in range(K // 16):
                iv = idx_v[pl.ds(eg + h * 16, 16)]
                wv = w_v[pl.ds(eg + h * 16, 16)]
                for t in range(16):
                    lj = iv[t] - s0
                    s = wv[t]
                    for v in range(8):
                        acc[v] = acc[v] + s * slab_v[lj, pl.ds(v * 16, 16)]
            for v in range(8):
                out_v[g, pl.ds(v * 16, 16)] = acc[v]
            return 0

        lax.fori_loop(0, chunk, dest, 0)
        pltpu.sync_copy(out_v, out_hbm.at[pl.ds(row0, chunk)])
        return 0

    lax.fori_loop(0, nper // chunk, do_chunk, 0)


def _nbr_sum(xf, jg, wg):
    rows, cdim = xf.shape
    nw = 32
    nper = rows // nw
    chunk = 32
    mesh = plsc.VectorSubcoreMesh(core_axis_name="c", subcore_axis_name="s")
    fn = functools.partial(
        pl.kernel,
        mesh=mesh,
        out_type=jax.ShapeDtypeStruct((rows, cdim), jnp.float32),
        scratch_types=[
            pltpu.VMEM((SLAB, cdim), jnp.float32),
            pltpu.VMEM((chunk * K,), jnp.int32),
            pltpu.VMEM((chunk * K,), jnp.float32),
            pltpu.VMEM((chunk, cdim), jnp.float32),
        ],
    )(functools.partial(_nbr_body, nper=nper, chunk=chunk, nbatch=N))
    return fn(xf, jg, wg)


def _mlp_body(x_ref, nbr_ref, w1_ref, b1_ref, w2_ref, b2_ref, out_ref):
    f32 = jnp.float32
    h = lax.dot_general(x_ref[...], w1_ref[...],
                        (((1,), (1,)), ((), ())),
                        preferred_element_type=f32)
    h = jnp.maximum(h + b1_ref[...], 0.0)
    y = lax.dot_general(h, w2_ref[...],
                        (((1,), (1,)), ((), ())),
                        preferred_element_type=f32)
    out_ref[...] = y + b2_ref[...] + nbr_ref[...] * (1.0 / K)


def _mlp_add(xf, nbr, w1, b1, w2, b2):
    rows, cdim = xf.shape
    blk = 1024
    grid = rows // blk
    return pl.pallas_call(
        _mlp_body,
        grid=(grid,),
        in_specs=[
            pl.BlockSpec((blk, cdim), lambda i: (i, 0)),
            pl.BlockSpec((blk, cdim), lambda i: (i, 0)),
            pl.BlockSpec(w1.shape, lambda i: (0, 0)),
            pl.BlockSpec(b1.shape, lambda i: (0, 0)),
            pl.BlockSpec(w2.shape, lambda i: (0, 0)),
            pl.BlockSpec(b2.shape, lambda i: (0, 0)),
        ],
        out_specs=pl.BlockSpec((blk, cdim), lambda i: (i, 0)),
        out_shape=jax.ShapeDtypeStruct((rows, cdim), jnp.float32),
    )(xf, nbr, w1, b1, w2, b2)


def kernel(x, W1_w, W1_b, W2_w, W2_b, phi_w, phi_i, phi_j,
           h1_w, h1_b, h2_w, h2_b, S_m_x, S_m_y):
    B, n, cdim = x.shape
    lin = jnp.linspace(0.0, 1.0, H)
    cx = jnp.repeat(lin, H).reshape(1, N)
    cy = jnp.tile(lin, H).reshape(1, N)
    zpad = jnp.zeros((1, PAD), dtype=jnp.float32)
    cxp = jnp.concatenate([zpad, cx, zpad], axis=1)
    cyp = jnp.concatenate([zpad, cy, zpad], axis=1)
    knots = jnp.linspace(0.0, 1.0, NKNOT).reshape(1, NKNOT)

    J, W = _select_weights(
        cxp, cyp, knots,
        S_m_x.reshape(1, NKNOT), S_m_y.reshape(1, NKNOT),
        phi_w.reshape(1, cdim), phi_i.T, phi_j.T)

    jf = J.T.reshape(-1)
    wf = W.T.reshape(-1)
    jg = jnp.concatenate([jf, jf + N])
    wg = jnp.concatenate([wf, wf])

    xf = x.reshape(B * n, cdim)
    nbr = _nbr_sum(xf, jg, wg)
    out = _mlp_add(xf, nbr, W1_w, W1_b.reshape(1, 2 * cdim),
                   W2_w, W2_b.reshape(1, cdim))
    return out.reshape(B, n, cdim)


# speedup vs baseline: 136.0790x; 1.0762x over previous
import functools

import jax
import jax.numpy as jnp
from jax import lax
from jax.experimental import pallas as pl
from jax.experimental.pallas import tpu as pltpu
from jax.experimental.pallas import tpu_sc as plsc

N = 4096
H = 64
K = 32
WIN = 5
WW = 2 * WIN + 1
NCAND = WW * WW
NCPAD = 128
PAD = WIN * H + WIN
NKNOT = 32
HALO = 328
SLAB = 256 + 2 * HALO


def _select_body(cxp_ref, cyp_ref, knots_ref, smx_ref, smy_ref,
                 phiw_ref, phii_ref, phij_ref, j_ref, w_ref):
    f32 = jnp.float32
    cx = cxp_ref[:, PAD:PAD + N]
    cy = cyp_ref[:, PAD:PAD + N]
    p = lax.broadcasted_iota(jnp.int32, (1, N), 1)
    r = p // H
    c = p % H

    rows = []
    inf_row = jnp.full((1, N), jnp.inf, dtype=f32)
    for o in range(NCAND):
        dr = o // WW - WIN
        dc = o % WW - WIN
        s = dr * H + dc
        cxs = cxp_ref[:, PAD + s:PAD + s + N]
        cys = cyp_ref[:, PAD + s:PAD + s + N]
        dx = cx - cxs
        dy = cy - cys
        dist = jnp.sqrt(dx * dx + dy * dy)
        u = r + dr
        v = c + dc
        ok = (u >= 0) & (u < H) & (v >= 0) & (v < H)
        rows.append(jnp.where(ok, dist, jnp.inf))
    for _ in range(NCPAD - NCAND):
        rows.append(inf_row)
    D = jnp.concatenate(rows, axis=0)
    ORD = lax.broadcasted_iota(jnp.int32, (NCPAD, N), 0)

    js = []
    for _ in range(K):
        m = jnp.min(D, axis=0, keepdims=True)
        wo = jnp.min(jnp.where(D == m, ORD, NCPAD), axis=0, keepdims=True)
        dr = wo // WW - WIN
        dc = wo % WW - WIN
        js.append(p + dr * H + dc)
        D = jnp.where(ORD == wo, jnp.inf, D)
    J = jnp.concatenate(js, axis=0)

    s390 = J - p + PAD
    DR = s390 // H - WIN
    DC = s390 % H - WIN
    inv = f32(1.0 / (H - 1))

    lane = lax.broadcasted_iota(jnp.int32, (1, 128), 1)
    relv = -(lane - WIN).astype(f32) * inv
    psixv = jnp.zeros((1, 128), dtype=f32)
    psiyv = jnp.zeros((1, 128), dtype=f32)
    for t in range(NKNOT):
        kt = knots_ref[0, t]
        hat = jnp.maximum(0.0, 1.0 - jnp.abs(relv - kt))
        psixv = psixv + hat * smx_ref[0, t]
        psiyv = psiyv + hat * smy_ref[0, t]
    DR5 = DR + WIN
    DC5 = DC + WIN
    psix = jnp.zeros((K, N), dtype=f32)
    psiy = jnp.zeros((K, N), dtype=f32)
    for l in range(WW):
        psix = psix + jnp.where(DR5 == l, psixv[0:1, l:l + 1], 0.0)
        psiy = psiy + jnp.where(DC5 == l, psiyv[0:1, l:l + 1], 0.0)
    psi = psix * psiy

    pw = phiw_ref[0:1, :]
    m00 = jnp.sum(pw * phii_ref[0:1, :] * phij_ref[0:1, :])
    m01 = jnp.sum(pw * phii_ref[0:1, :] * phij_ref[1:2, :])
    m10 = jnp.sum(pw * phii_ref[1:2, :] * phij_ref[0:1, :])
    m11 = jnp.sum(pw * phii_ref[1:2, :] * phij_ref[1:2, :])
    qs = cx * (m00 * cx + m01 * cy) + cy * (m10 * cx + m11 * cy)
    gx = (cx * m00 + cy * m10) * inv
    gy = (cx * m01 + cy * m11) * inv
    q = qs + gx * DR.astype(f32) + gy * DC.astype(f32)

    nedge = f32(1.0 / (K * N))
    mp = jnp.sum(jnp.abs(psi)) * nedge + 1e-6
    mq = jnp.sum(jnp.abs(q)) * nedge + 1e-6

    j_ref[...] = J
    w_ref[...] = psi * q * (1.0 / (mp * mq))


def _select_weights(cxp, cyp, knots, smx, smy, phiw, phii2, phij2):
    return pl.pallas_call(
        _select_body,
        out_shape=(
            jax.ShapeDtypeStruct((K, N), jnp.int32),
            jax.ShapeDtypeStruct((K, N), jnp.float32),
        ),
    )(cxp, cyp, knots, smx, smy, phiw, phii2, phij2)


def _nbr_body(xf_hbm, jg_hbm, wg_hbm, out_hbm, slab_v, idx_v, w_v, out_v,
              *, nper, chunk, nbatch):
    wid = lax.axis_index("c") * 16 + lax.axis_index("s")
    base = wid * nper
    p0 = lax.rem(base, nbatch)
    b0 = base - p0
    s0 = b0 + jnp.clip(p0 - HALO, 0, nbatch - SLAB)
    s0 = pl.multiple_of(s0, 8)
    pltpu.sync_copy(xf_hbm.at[pl.ds(s0, SLAB)], slab_v)
    ek = chunk * K

    def do_chunk(ci, _):
        row0 = base + ci * chunk
        e0 = row0 * K
        pltpu.sync_copy(jg_hbm.at[pl.ds(e0, ek)], idx_v)
        pltpu.sync_copy(wg_hbm.at[pl.ds(e0, ek)], w_v)

        def dest(g, _g):
            eg = g * K
            acc = [jnp.zeros((16,), jnp.float32) for _ in range(8)]
            for h in range(K // 16):
                iv = idx_v[pl.ds(eg + h * 16, 16)]
                wv = w_v[pl.ds(eg + h * 16, 16)]
                for t in range(16):
                    lj = iv[t] - s0
                    s = wv[t]
                    for v in range(8):
                        acc[v] = acc[v] + s * slab_v[lj, pl.ds(v * 16, 16)]
            for v in range(8):
                out_v[g, pl.ds(v * 16, 16)] = acc[v]
            return 0

        lax.fori_loop(0, chunk, dest, 0)
        pltpu.sync_copy(out_v, out_hbm.at[pl.ds(row0, chunk)])
        return 0

    lax.fori_loop(0, nper // chunk, do_chunk, 0)


def _nbr_sum(xf, jg, wg):
    rows, cdim = xf.shape
    nw = 32
    nper = rows // nw
    chunk = 32
    mesh = plsc.VectorSubcoreMesh(core_axis_name="c", subcore_axis_name="s")
    fn = functools.partial(
        pl.kernel,
        mesh=mesh,
        out_type=jax.ShapeDtypeStruct((rows, cdim), jnp.float32),
        scratch_types=[
            pltpu.VMEM((SLAB, cdim), jnp.float32),
            pltpu.VMEM((chunk * K,), jnp.int32),
            pltpu.VMEM((chunk * K,), jnp.float32),
            pltpu.VMEM((chunk, cdim), jnp.float32),
        ],
    )(functools.partial(_nbr_body, nper=nper, chunk=chunk, nbatch=N))
    return fn(xf, jg, wg)


def _mlp_body(x_ref, nbr_ref, w1_ref, b1_ref, w2_ref, b2_ref, out_ref):
    f32 = jnp.float32
    h = lax.dot_general(x_ref[...], w1_ref[...],
                        (((1,), (1,)), ((), ())),
                        preferred_element_type=f32)
    h = jnp.maximum(h + b1_ref[...], 0.0)
    y = lax.dot_general(h, w2_ref[...],
                        (((1,), (1,)), ((), ())),
                        preferred_element_type=f32)
    out_ref[...] = y + b2_ref[...] + nbr_ref[...] * (1.0 / K)


def _mlp_add(xf, nbr, w1, b1, w2, b2):
    rows, cdim = xf.shape
    blk = 1024
    grid = rows // blk
    return pl.pallas_call(
        _mlp_body,
        grid=(grid,),
        in_specs=[
            pl.BlockSpec((blk, cdim), lambda i: (i, 0)),
            pl.BlockSpec((blk, cdim), lambda i: (i, 0)),
            pl.BlockSpec(w1.shape, lambda i: (0, 0)),
            pl.BlockSpec(b1.shape, lambda i: (0, 0)),
            pl.BlockSpec(w2.shape, lambda i: (0, 0)),
            pl.BlockSpec(b2.shape, lambda i: (0, 0)),
        ],
        out_specs=pl.BlockSpec((blk, cdim), lambda i: (i, 0)),
        out_shape=jax.ShapeDtypeStruct((rows, cdim), jnp.float32),
    )(xf, nbr, w1, b1, w2, b2)


def kernel(x, W1_w, W1_b, W2_w, W2_b, phi_w, phi_i, phi_j,
           h1_w, h1_b, h2_w, h2_b, S_m_x, S_m_y):
    B, n, cdim = x.shape
    lin = jnp.linspace(0.0, 1.0, H)
    cx = jnp.repeat(lin, H).reshape(1, N)
    cy = jnp.tile(lin, H).reshape(1, N)
    zpad = jnp.zeros((1, PAD), dtype=jnp.float32)
    cxp = jnp.concatenate([zpad, cx, zpad], axis=1)
    cyp = jnp.concatenate([zpad, cy, zpad], axis=1)
    knots = jnp.linspace(0.0, 1.0, NKNOT).reshape(1, NKNOT)

    J, W = _select_weights(
        cxp, cyp, knots,
        S_m_x.reshape(1, NKNOT), S_m_y.reshape(1, NKNOT),
        phi_w.reshape(1, cdim), phi_i.T, phi_j.T)

    jf = J.T.reshape(-1)
    wf = W.T.reshape(-1)
    jg = jnp.concatenate([jf, jf + N])
    wg = jnp.concatenate([wf, wf])

    xf = x.reshape(B * n, cdim)
    nbr = _nbr_sum(xf, jg, wg)
    out = _mlp_add(xf, nbr, W1_w, W1_b.reshape(1, 2 * cdim),
                   W2_w, W2_b.reshape(1, cdim))
    return out.reshape(B, n, cdim)
